# baseline (speedup 1.0000x reference)
import functools

import jax
import jax.numpy as jnp
from jax.experimental import pallas as pl


def _batch_body(nt, inv_scale, x_ref, w_ref, b_ref, o_ref):
    w = w_ref[...]
    bias = b_ref[0]
    for t in range(nt):
        xi = x_ref[0, :, t, :]
        xb = xi.astype(jnp.bfloat16)
        xs = xb * jnp.bfloat16(inv_scale)
        s = jax.lax.dot_general(
            xs, xb, (((1,), (1,)), ((), ())),
            preferred_element_type=jnp.float32)
        s = jnp.maximum(s, 0.0)
        m = jnp.max(s, axis=1, keepdims=True)
        e = jnp.exp(s - m)
        denom = jnp.sum(e, axis=1, keepdims=True)
        hh = jnp.dot(e.astype(jnp.bfloat16), xb,
                     preferred_element_type=jnp.float32)
        hh = hh / denom
        hh = jnp.dot(hh.astype(jnp.bfloat16), w,
                     preferred_element_type=jnp.float32)
        o_ref[0, :, t, :] = jnp.maximum(hh + bias, 0.0)


def kernel(x, W, b):
    Bx, N, T, D = x.shape
    H = W.shape[1]
    Wb = W.astype(jnp.bfloat16)
    b2 = b.reshape(1, H)
    inv_scale = 1.0 / float(D) ** 0.5

    out = pl.pallas_call(
        functools.partial(_batch_body, T, inv_scale),
        grid=(Bx,),
        in_specs=[
            pl.BlockSpec((1, N, T, D), lambda bb: (bb, 0, 0, 0)),
            pl.BlockSpec((D, H), lambda bb: (0, 0)),
            pl.BlockSpec((1, H), lambda bb: (0, 0)),
        ],
        out_specs=pl.BlockSpec((1, N, T, H), lambda bb: (bb, 0, 0, 0)),
        out_shape=jax.ShapeDtypeStruct((Bx, N, T, H), jnp.float32),
    )(x, Wb, b2)
    return out


# speedup vs baseline: 1.0240x; 1.0240x over previous
import functools

import jax
import jax.numpy as jnp
from jax.experimental import pallas as pl
from jax.experimental.pallas import tpu as pltpu


def _batch_body(nt, inv_scale, x_ref, w_ref, b_ref, o_ref,
                ibuf, obuf, isem, osem):
    for t in range(nt):
        pltpu.make_async_copy(x_ref.at[0, :, t, :], ibuf.at[t],
                              isem.at[t]).start()
    w = w_ref[...]
    bias = b_ref[0]
    for t in range(nt):
        pltpu.make_async_copy(x_ref.at[0, :, t, :], ibuf.at[t],
                              isem.at[t]).wait()
        xi = ibuf[t]
        xb = xi.astype(jnp.bfloat16)
        xs = xb * jnp.bfloat16(inv_scale)
        s = jax.lax.dot_general(
            xs, xb, (((1,), (1,)), ((), ())),
            preferred_element_type=jnp.float32)
        s = jnp.maximum(s, 0.0)
        m = jnp.max(s, axis=1, keepdims=True)
        e = jnp.exp(s - m)
        denom = jnp.sum(e, axis=1, keepdims=True)
        hh = jnp.dot(e.astype(jnp.bfloat16), xb,
                     preferred_element_type=jnp.float32)
        hh = hh / denom
        hh = jnp.dot(hh.astype(jnp.bfloat16), w,
                     preferred_element_type=jnp.float32)
        obuf[t] = jnp.maximum(hh + bias, 0.0)
        pltpu.make_async_copy(obuf.at[t], o_ref.at[0, :, t, :],
                              osem.at[t]).start()
    for t in range(nt):
        pltpu.make_async_copy(obuf.at[t], o_ref.at[0, :, t, :],
                              osem.at[t]).wait()


def kernel(x, W, b):
    Bx, N, T, D = x.shape
    H = W.shape[1]
    Wb = W.astype(jnp.bfloat16)
    b2 = b.reshape(1, H)
    inv_scale = 1.0 / float(D) ** 0.5

    out = pl.pallas_call(
        functools.partial(_batch_body, T, inv_scale),
        grid=(Bx,),
        in_specs=[
            pl.BlockSpec((1, N, T, D), lambda bb: (bb, 0, 0, 0)),
            pl.BlockSpec((D, H), lambda bb: (0, 0)),
            pl.BlockSpec((1, H), lambda bb: (0, 0)),
        ],
        out_specs=pl.BlockSpec((1, N, T, H), lambda bb: (bb, 0, 0, 0)),
        out_shape=jax.ShapeDtypeStruct((Bx, N, T, H), jnp.float32),
        scratch_shapes=[
            pltpu.VMEM((T, N, D), jnp.float32),
            pltpu.VMEM((T, N, H), jnp.float32),
            pltpu.SemaphoreType.DMA((T,)),
            pltpu.SemaphoreType.DMA((T,)),
        ],
    )(x, Wb, b2)
    return out


# speedup vs baseline: 1.1514x; 1.1244x over previous
import functools

import jax
import jax.numpy as jnp
from jax.experimental import pallas as pl


def _batch_body(nt, inv_scale, x_ref, w_ref, b_ref, o_ref):
    w = w_ref[...]
    bias = b_ref[0]
    d = w.shape[0]
    xall = x_ref[0]
    for t in range(nt):
        xi = xall[:, t * d:(t + 1) * d]
        xb = xi.astype(jnp.bfloat16)
        xs = xb * jnp.bfloat16(inv_scale)
        s = jax.lax.dot_general(
            xs, xb, (((1,), (1,)), ((), ())),
            preferred_element_type=jnp.float32)
        s = jnp.maximum(s, 0.0)
        m = jnp.max(s, axis=1, keepdims=True)
        e = jnp.exp(s - m)
        denom = jnp.sum(e, axis=1, keepdims=True)
        hh = jnp.dot(e.astype(jnp.bfloat16), xb,
                     preferred_element_type=jnp.float32)
        hh = hh / denom
        hh = jnp.dot(hh.astype(jnp.bfloat16), w,
                     preferred_element_type=jnp.float32)
        o_ref[0, :, t, :] = jnp.maximum(hh + bias, 0.0)


def kernel(x, W, b):
    Bx, N, T, D = x.shape
    H = W.shape[1]
    x2 = x.reshape(Bx, N, T * D)
    Wb = W.astype(jnp.bfloat16)
    b2 = b.reshape(1, H)
    inv_scale = 1.0 / float(D) ** 0.5

    out = pl.pallas_call(
        functools.partial(_batch_body, T, inv_scale),
        grid=(Bx,),
        in_specs=[
            pl.BlockSpec((1, N, T * D), lambda bb: (bb, 0, 0)),
            pl.BlockSpec((D, H), lambda bb: (0, 0)),
            pl.BlockSpec((1, H), lambda bb: (0, 0)),
        ],
        out_specs=pl.BlockSpec((1, N, T, H), lambda bb: (bb, 0, 0, 0)),
        out_shape=jax.ShapeDtypeStruct((Bx, N, T, H), jnp.float32),
    )(x2, Wb, b2)
    return out
